# Initial kernel scaffold; baseline (speedup 1.0000x reference)
import jax
import jax.numpy as jnp
from jax.experimental import pallas as pl


def kernel(pos_u, pos_v, neg_v, U, V):
    raise NotImplementedError("write your pallas kernel here")


# speedup vs baseline: 5.2651x; 5.2651x over previous
import functools

import jax
import jax.numpy as jnp
from jax import lax
from jax.experimental import pallas as pl
from jax.experimental.pallas import tpu as pltpu
from jax.experimental.pallas import tpu_sc as plsc

_B = 16384
_D = 64
_NEG = 20
_NC = 2
_NS = 16
_NW = _NC * _NS
_BW = _B // _NW
_C = 64
_NCH = _BW // _C
_SEG = _C * (1 + _NEG)
_NSCORE = _B * (1 + _NEG)


def _sc_scores(pos_u, pos_v, neg_r, U, V):
    mesh = plsc.VectorSubcoreMesh(core_axis_name="c", subcore_axis_name="s")

    @functools.partial(
        pl.kernel,
        out_type=jax.ShapeDtypeStruct((_NSCORE,), jnp.float32),
        mesh=mesh,
        compiler_params=pltpu.CompilerParams(needs_layout_passes=False,
                                             use_tc_tiling_on_sc=False),
        scratch_types=[
            pltpu.VMEM((_C,), jnp.int32),
            pltpu.VMEM((_C,), jnp.int32),
            pltpu.VMEM((_NEG * _C,), jnp.int32),
            pltpu.VMEM((_C, _D), jnp.float32),
            pltpu.VMEM((_C, _D), jnp.float32),
            pltpu.VMEM((_NEG * _C, _D), jnp.float32),
            pltpu.VMEM((_C,), jnp.float32),
            pltpu.VMEM((_NEG * _C + 16,), jnp.float32),
            pltpu.SemaphoreType.DMA,
        ],
    )
    def k(pos_u_h, pos_v_h, neg_r_h, U_h, V_h, out_h,
          idx_u, idx_v, idx_n, u_rows, v_rows, n_rows, sc_pos, sc_neg, sem):
        wid = lax.axis_index("s") * _NC + lax.axis_index("c")
        li = lax.broadcasted_iota(jnp.int32, (16,), 0)

        def chunk_body(c, _):
            seg = wid * _NCH + c
            cbase = seg * _C
            pltpu.sync_copy(pos_u_h.at[pl.ds(cbase, _C)], idx_u)
            pltpu.sync_copy(pos_v_h.at[pl.ds(cbase, _C)], idx_v)
            pltpu.sync_copy(neg_r_h.at[pl.ds(seg * _NEG * _C, _NEG * _C)],
                            idx_n)
            cps = [
                pltpu.async_copy(U_h.at[idx_u], u_rows, sem),
                pltpu.async_copy(V_h.at[idx_v], v_rows, sem),
            ]
            for j in range(_NEG):
                cps.append(pltpu.async_copy(
                    V_h.at[idx_n.at[pl.ds(j * _C, _C)]],
                    n_rows.at[pl.ds(j * _C, _C)], sem))
            for cp in cps:
                cp.wait()

            def pos_body(g, _):
                acc = jnp.zeros((16,), jnp.float32)
                for kk in range(16):
                    b = g * 16 + kk
                    p = (u_rows[b, pl.ds(0, 16)] * v_rows[b, pl.ds(0, 16)]
                         + u_rows[b, pl.ds(16, 16)] * v_rows[b, pl.ds(16, 16)]
                         + u_rows[b, pl.ds(32, 16)] * v_rows[b, pl.ds(32, 16)]
                         + u_rows[b, pl.ds(48, 16)] * v_rows[b, pl.ds(48, 16)])
                    acc = jnp.where(li == kk, jnp.sum(p), acc)
                sc_pos[pl.ds(g * 16, 16)] = acc
                return 0

            lax.fori_loop(0, _C // 16, pos_body, 0)

            def neg_body(b, _):
                u0 = u_rows[b, pl.ds(0, 16)]
                u1 = u_rows[b, pl.ds(16, 16)]
                u2 = u_rows[b, pl.ds(32, 16)]
                u3 = u_rows[b, pl.ds(48, 16)]
                acc1 = jnp.zeros((16,), jnp.float32)
                acc2 = jnp.zeros((16,), jnp.float32)
                for j in range(_NEG):
                    r = j * _C + b
                    p = (n_rows[r, pl.ds(0, 16)] * u0
                         + n_rows[r, pl.ds(16, 16)] * u1
                         + n_rows[r, pl.ds(32, 16)] * u2
                         + n_rows[r, pl.ds(48, 16)] * u3)
                    s = -jnp.sum(p)
                    if j < 16:
                        acc1 = jnp.where(li == j, s, acc1)
                    else:
                        acc2 = jnp.where(li == (j - 16), s, acc2)
                sc_neg[pl.ds(b * _NEG, 16)] = acc1
                tail = sc_neg[pl.ds(b * _NEG + 16, 16)]
                sc_neg[pl.ds(b * _NEG + 16, 16)] = jnp.where(li < 4, acc2,
                                                             tail)
                return 0

            lax.fori_loop(0, _C, neg_body, 0)

            obase = seg * _SEG
            pltpu.sync_copy(sc_pos, out_h.at[pl.ds(obase, _C)])
            pltpu.sync_copy(sc_neg.at[pl.ds(0, _NEG * _C)],
                            out_h.at[pl.ds(obase + _C, _NEG * _C)])
            return 0

        lax.fori_loop(0, _NCH, chunk_body, 0)

    return k(pos_u, pos_v, neg_r, U, V)


def _tc_logsig_sum(x2d):
    def body(x_ref, o_ref):
        x = x_ref[...]
        ls = jnp.minimum(x, 0.0) - jnp.log1p(jnp.exp(-jnp.abs(x)))
        o_ref[0, 0] = -jnp.sum(ls)

    return pl.pallas_call(
        body,
        out_shape=jax.ShapeDtypeStruct((1, 1), jnp.float32),
        out_specs=pl.BlockSpec(memory_space=pltpu.SMEM),
    )(x2d)


def kernel(pos_u, pos_v, neg_v, U, V):
    neg_r = neg_v.reshape(_NW * _NCH, _C, _NEG).transpose(0, 2, 1).reshape(-1)
    scores = _sc_scores(pos_u, pos_v, neg_r, U, V)
    res = _tc_logsig_sum(scores.reshape(_NSCORE // 128, 128))
    return res[0, 0]
